# baseline (speedup 1.0000x reference)
import functools

import jax
import jax.numpy as jnp
from jax import lax
from jax.experimental import pallas as pl
from jax.experimental.pallas import tpu as pltpu
from jax.experimental.pallas import tpu_sc as plsc

EMBED_DIM = 64
NUM_CORES = 2
NUM_SUBCORES = 16
NUM_WORKERS = NUM_CORES * NUM_SUBCORES

B_TOTAL = 4096 * 200
B_PER_W = B_TOTAL // NUM_WORKERS
CHUNK = 512
N_CHUNKS = B_PER_W // CHUNK
NBUF = 3

_mesh = plsc.VectorSubcoreMesh(core_axis_name="c", subcore_axis_name="s")

_scratch = (
    [pltpu.VMEM((B_PER_W,), jnp.int32)]
    + [pltpu.VMEM((CHUNK, EMBED_DIM), jnp.float32) for _ in range(NBUF)]
    + [pltpu.SemaphoreType.DMA for _ in range(2 * NBUF)]
)


@functools.partial(
    pl.kernel,
    mesh=_mesh,
    out_type=jax.ShapeDtypeStruct((B_TOTAL, EMBED_DIM), jnp.float32),
    compiler_params=pltpu.CompilerParams(use_tc_tiling_on_sc=False),
    scratch_types=_scratch,
)
def _gather_kernel(idx_hbm, table_hbm, out_hbm, idx_all, *bufs_sems):
    row_bufs = bufs_sems[:NBUF]
    gsems = bufs_sems[NBUF:2 * NBUF]
    osems = bufs_sems[2 * NBUF:]

    wid = lax.axis_index("s") * NUM_CORES + lax.axis_index("c")
    base = wid * B_PER_W

    pltpu.sync_copy(idx_hbm.at[pl.ds(base, B_PER_W)], idx_all)

    def gather_start(i, b):
        return pltpu.async_copy(
            table_hbm.at[idx_all.at[pl.ds(i * CHUNK, CHUNK)]],
            row_bufs[b], gsems[b])

    def out_start(i, b):
        return pltpu.async_copy(
            row_bufs[b], out_hbm.at[pl.ds(base + i * CHUNK, CHUNK)],
            osems[b])

    def gather_wait(b):
        pltpu.make_async_copy(
            out_hbm.at[pl.ds(base, CHUNK)], row_bufs[b], gsems[b]).wait()

    def out_wait(b):
        pltpu.make_async_copy(
            out_hbm.at[pl.ds(base, CHUNK)], row_bufs[b], osems[b]).wait()

    for b in range(NBUF):
        gather_start(b, b)

    def body(g, _):
        for b in range(NBUF):
            i = g * NBUF + b

            @pl.when(i < N_CHUNKS)
            def _():
                gather_wait(b)
                out_start(i, b)
                nxt = i + NBUF

                @pl.when(nxt < N_CHUNKS)
                def _():
                    out_wait(b)
                    gather_start(nxt, b)
        return 0

    n_outer = (N_CHUNKS + NBUF - 1) // NBUF
    lax.fori_loop(0, n_outer, body, 0)

    for b in range(NBUF):
        out_wait(b)


def kernel(input, table):
    idx = input.reshape(-1).astype(jnp.int32)
    out = _gather_kernel(idx, table)
    return out.reshape(input.shape + (EMBED_DIM,))


# speedup vs baseline: 1.0241x; 1.0241x over previous
import functools

import jax
import jax.numpy as jnp
from jax import lax
from jax.experimental import pallas as pl
from jax.experimental.pallas import tpu as pltpu
from jax.experimental.pallas import tpu_sc as plsc

EMBED_DIM = 64
BATCH = 4096
HIST = 200
NUM_CORES = 2
NUM_SUBCORES = 16
NUM_WORKERS = NUM_CORES * NUM_SUBCORES

B_BLK = 512
ROWG = B_BLK // 8
BLKS_PER_H = BATCH // B_BLK
N_TASKS = HIST * BLKS_PER_H
TASKS_PER_W = N_TASKS // NUM_WORKERS

_mesh = plsc.VectorSubcoreMesh(core_axis_name="c", subcore_axis_name="s")

_scratch = (
    [pltpu.VMEM((B_BLK,), jnp.int32) for _ in range(2)]
    + [pltpu.VMEM((ROWG, 8, EMBED_DIM), jnp.float32) for _ in range(2)]
    + [pltpu.SemaphoreType.DMA for _ in range(6)]
)


@functools.partial(
    pl.kernel,
    mesh=_mesh,
    out_type=jax.ShapeDtypeStruct((HIST, BATCH, EMBED_DIM), jnp.float32),
    compiler_params=pltpu.CompilerParams(use_tc_tiling_on_sc=False),
    scratch_types=_scratch,
)
def _lookup_kernel(idxt_hbm, table_hbm, out_hbm,
                   idx0, idx1, rows0, rows1,
                   isem0, isem1, gsem0, gsem1, osem0, osem1):
    idx_bufs = (idx0, idx1)
    row_bufs = (rows0, rows1)
    isems = (isem0, isem1)
    gsems = (gsem0, gsem1)
    osems = (osem0, osem1)

    wid = lax.axis_index("s") * NUM_CORES + lax.axis_index("c")

    def task_of(i):
        t = wid + i * NUM_WORKERS
        return t // BLKS_PER_H, (t % BLKS_PER_H) * B_BLK

    def idx_start(i, s):
        h, b0 = task_of(i)
        pltpu.async_copy(idxt_hbm.at[h, pl.ds(b0, B_BLK)], idx_bufs[s],
                         isems[s])

    def idx_wait(s):
        pltpu.make_async_copy(
            idxt_hbm.at[0, pl.ds(0, B_BLK)], idx_bufs[s], isems[s]).wait()

    def gather_start(s):
        for j in range(ROWG):
            pltpu.async_copy(table_hbm.at[idx_bufs[s].at[pl.ds(j * 8, 8)]],
                             row_bufs[s].at[j], gsems[s])

    def gather_wait(s):
        for _ in range(ROWG):
            pltpu.make_async_copy(table_hbm.at[pl.ds(0, 8)],
                                  row_bufs[s].at[0], gsems[s]).wait()

    def out_start(i, s):
        h, b0 = task_of(i)
        for j in range(ROWG):
            pltpu.async_copy(
                row_bufs[s].at[j],
                out_hbm.at[h, pl.ds(b0 + j * 8, 8)],
                osems[s])

    def out_wait(s):
        for _ in range(ROWG):
            pltpu.make_async_copy(
                out_hbm.at[0, pl.ds(0, 8)],
                row_bufs[s].at[0], osems[s]).wait()

    idx_start(0, 0)
    idx_wait(0)
    gather_start(0)
    idx_start(1, 1)
    idx_wait(1)
    gather_start(1)

    def body(g, _):
        for s in range(2):
            i = g * 2 + s
            gather_wait(s)
            out_start(i, s)

            @pl.when(i + 2 < TASKS_PER_W)
            def _():
                idx_start(i + 2, s)
                idx_wait(s)
                out_wait(s)
                gather_start(s)
        return 0

    lax.fori_loop(0, TASKS_PER_W // 2, body, 0)
    out_wait(0)
    out_wait(1)


def kernel(input, table):
    out3 = _lookup_kernel(input.T, table)
    return jnp.swapaxes(out3, 0, 1)


# speedup vs baseline: 1.0307x; 1.0064x over previous
import functools

import jax
import jax.numpy as jnp
from jax import lax
from jax.experimental import pallas as pl
from jax.experimental.pallas import tpu as pltpu
from jax.experimental.pallas import tpu_sc as plsc

EMBED_DIM = 64
BATCH = 4096
HIST = 200
NUM_CORES = 2
NUM_SUBCORES = 16
NUM_WORKERS = NUM_CORES * NUM_SUBCORES

B_BLK = 512
ROWG = B_BLK // 8
BLKS_PER_H = BATCH // B_BLK
N_TASKS = HIST * BLKS_PER_H
TASKS_PER_W = N_TASKS // NUM_WORKERS

_mesh = plsc.VectorSubcoreMesh(core_axis_name="c", subcore_axis_name="s")

_scratch = (
    [pltpu.VMEM((B_BLK,), jnp.int32) for _ in range(2)]
    + [pltpu.VMEM((B_BLK, EMBED_DIM), jnp.float32) for _ in range(2)]
    + [pltpu.SemaphoreType.DMA for _ in range(6)]
)


@functools.partial(
    pl.kernel,
    mesh=_mesh,
    out_type=jax.ShapeDtypeStruct((HIST, BATCH, EMBED_DIM), jnp.float32),
    compiler_params=pltpu.CompilerParams(use_tc_tiling_on_sc=False),
    scratch_types=_scratch,
)
def _lookup_kernel(idxt_hbm, table_hbm, out_hbm,
                   idx0, idx1, rows0, rows1,
                   isem0, isem1, gsem0, gsem1, osem0, osem1):
    idx_bufs = (idx0, idx1)
    row_bufs = (rows0, rows1)
    isems = (isem0, isem1)
    gsems = (gsem0, gsem1)
    osems = (osem0, osem1)

    wid = lax.axis_index("s") * NUM_CORES + lax.axis_index("c")

    def task_of(i):
        t = wid + i * NUM_WORKERS
        return t // BLKS_PER_H, (t % BLKS_PER_H) * B_BLK

    def idx_start(i, s):
        h, b0 = task_of(i)
        pltpu.async_copy(idxt_hbm.at[h, pl.ds(b0, B_BLK)], idx_bufs[s],
                         isems[s])

    def idx_wait(s):
        pltpu.make_async_copy(
            idxt_hbm.at[0, pl.ds(0, B_BLK)], idx_bufs[s], isems[s]).wait()

    def gather_start(s):
        pltpu.async_copy(table_hbm.at[idx_bufs[s]], row_bufs[s], gsems[s])

    def gather_wait(s):
        pltpu.make_async_copy(table_hbm.at[pl.ds(0, B_BLK)],
                              row_bufs[s], gsems[s]).wait()

    def out_start(i, s):
        h, b0 = task_of(i)
        pltpu.async_copy(row_bufs[s], out_hbm.at[h, pl.ds(b0, B_BLK)],
                         osems[s])

    def out_wait(s):
        pltpu.make_async_copy(
            out_hbm.at[0, pl.ds(0, B_BLK)], row_bufs[s], osems[s]).wait()

    idx_start(0, 0)
    idx_wait(0)
    gather_start(0)
    idx_start(1, 1)
    idx_wait(1)
    gather_start(1)

    def body(g, _):
        for s in range(2):
            i = g * 2 + s
            gather_wait(s)
            out_start(i, s)

            @pl.when(i + 2 < TASKS_PER_W)
            def _():
                idx_start(i + 2, s)
                idx_wait(s)
                out_wait(s)
                gather_start(s)
        return 0

    lax.fori_loop(0, TASKS_PER_W // 2, body, 0)
    out_wait(0)
    out_wait(1)


def kernel(input, table):
    out3 = _lookup_kernel(input.T, table)
    return jnp.swapaxes(out3, 0, 1)


# speedup vs baseline: 1.0336x; 1.0028x over previous
import functools

import jax
import jax.numpy as jnp
from jax import lax
from jax.experimental import pallas as pl
from jax.experimental.pallas import tpu as pltpu
from jax.experimental.pallas import tpu_sc as plsc

EMBED_DIM = 64
BATCH = 4096
HIST = 200
NUM_CORES = 2
NUM_SUBCORES = 16
NUM_WORKERS = NUM_CORES * NUM_SUBCORES

B_BLK = 512
BLKS_PER_H = BATCH // B_BLK
N_TASKS = HIST * BLKS_PER_H
TASKS_PER_W = N_TASKS // NUM_WORKERS

_mesh = plsc.VectorSubcoreMesh(core_axis_name="c", subcore_axis_name="s")

_scratch = (
    [pltpu.VMEM((B_BLK,), jnp.int32) for _ in range(2)]
    + [pltpu.VMEM((B_BLK, EMBED_DIM), jnp.float32) for _ in range(2)]
    + [pltpu.SemaphoreType.DMA for _ in range(6)]
)


@functools.partial(
    pl.kernel,
    mesh=_mesh,
    out_type=jax.ShapeDtypeStruct((HIST, BATCH, EMBED_DIM), jnp.float32),
    compiler_params=pltpu.CompilerParams(use_tc_tiling_on_sc=False),
    scratch_types=_scratch,
)
def _lookup_kernel(idxt_hbm, table_hbm, out_hbm,
                   idx0, idx1, rows0, rows1,
                   isem0, isem1, gsem0, gsem1, osem0, osem1):
    idx_bufs = (idx0, idx1)
    row_bufs = (rows0, rows1)
    isems = (isem0, isem1)
    gsems = (gsem0, gsem1)
    osems = (osem0, osem1)

    wid = lax.axis_index("s") * NUM_CORES + lax.axis_index("c")

    def task_of(i):
        t = wid + i * NUM_WORKERS
        return t // BLKS_PER_H, (t % BLKS_PER_H) * B_BLK

    def idx_start(i, s):
        h, b0 = task_of(i)
        pltpu.async_copy(idxt_hbm.at[h, pl.ds(b0, B_BLK)], idx_bufs[s],
                         isems[s])

    def idx_wait(s):
        pltpu.make_async_copy(
            idxt_hbm.at[0, pl.ds(0, B_BLK)], idx_bufs[s], isems[s]).wait()

    def gather_start(s):
        pltpu.async_copy(table_hbm.at[idx_bufs[s]], row_bufs[s], gsems[s])

    def gather_wait(s):
        pltpu.make_async_copy(table_hbm.at[pl.ds(0, B_BLK)],
                              row_bufs[s], gsems[s]).wait()

    def out_start(i, s):
        h, b0 = task_of(i)
        pltpu.async_copy(row_bufs[s], out_hbm.at[h, pl.ds(b0, B_BLK)],
                         osems[s])

    def out_wait(s):
        pltpu.make_async_copy(
            out_hbm.at[0, pl.ds(0, B_BLK)], row_bufs[s], osems[s]).wait()

    idx_start(0, 0)
    idx_wait(0)
    gather_start(0)
    idx_start(1, 1)
    idx_wait(1)
    gather_start(1)

    def body(g, _):
        for s in range(2):
            i = g * 2 + s
            gather_wait(s)
            out_start(i, s)

            @pl.when(i + 2 < TASKS_PER_W)
            def _():
                idx_start(i + 2, s)
                idx_wait(s)
                out_wait(s)
                gather_start(s)
        return 0

    lax.fori_loop(0, TASKS_PER_W // 2, body, 0)
    out_wait(0)
    out_wait(1)


def kernel(input, table):
    out3 = _lookup_kernel(input.T, table)
    return jnp.swapaxes(out3, 0, 1)
